# baseline (speedup 1.0000x reference)
import jax
import jax.numpy as jnp
from jax import lax
from jax.experimental import pallas as pl
from jax.experimental.pallas import tpu as pltpu
from jax.experimental.pallas import tpu_sc as plsc

B = 16384
EMBED = 32
NEG = 8
NC = 2
NS = 16
NW = NC * NS
N = 1000000
NALIGNED = 999936
CH = 512
CPW = NALIGNED // (NW * CH)
NPACK = N // 4
BPW = B // NW
C = 64
NCHUNK = BPW // C
H = 16


def _relayout_body(u_t, i_t, u_tail, i_tail, u_pack, i_pack,
                   chunk, out, tailbuf):
    wid = lax.axis_index("s") * NC + lax.axis_index("c")
    lanes = lax.broadcasted_iota(jnp.int32, (H,), 0)

    for tbl, pack, tail in ((u_t, u_pack, u_tail), (i_t, i_pack, i_tail)):
        def chunk_body(c, _):
            col0 = (wid * CPW + c) * CH
            pltpu.sync_copy(tbl.at[:, pl.ds(col0, CH)],
                            chunk.at[:, pl.ds(0, CH)])

            def rb(r4, _):
                for r4q in range(16):
                    r = r4 * 4 + r4q // 4
                    q = r4q % 4
                    col = jnp.full((H,), 4 * r + q, jnp.int32)
                    for h in range(2):
                        v = plsc.load_gather(chunk, [h * H + lanes, col])
                        out[r, pl.ds(q * 32 + h * H, H)] = v
                return 0

            lax.fori_loop(0, CH // 16, rb, 0)
            pltpu.sync_copy(
                out, pack.at[pl.ds(pl.multiple_of(col0 // 4, 8), CH // 4), :])
            return 0

        lax.fori_loop(0, CPW, chunk_body, 0)

        @pl.when(wid == 0)
        def _():
            pltpu.sync_copy(tail, tailbuf)
            pltpu.sync_copy(tailbuf, pack.at[pl.ds(NALIGNED // 4, 16), :])


_relayout = pl.kernel(
    _relayout_body,
    out_type=(
        jax.ShapeDtypeStruct((NPACK, 128), jnp.float32),
        jax.ShapeDtypeStruct((NPACK, 128), jnp.float32),
    ),
    mesh=plsc.VectorSubcoreMesh(
        core_axis_name="c", subcore_axis_name="s",
        num_cores=NC, num_subcores=NS),
    scratch_types=[
        pltpu.VMEM((EMBED, CH + 8), jnp.float32),
        pltpu.VMEM((CH // 4, 128), jnp.float32),
        pltpu.VMEM((16, 128), jnp.float32),
    ],
    compiler_params=pltpu.CompilerParams(needs_layout_passes=False),
)


def _mf_body(u_pack, i_pack, uq_hbm, uo_hbm, iq_hbm, io_hbm, nq_hbm, no_hbm,
             pos_hbm, neg_hbm,
             uq, uo, iq, io, nq, no, u_rows, it_rows, ineg_rows,
             pos_buf, neg_buf, sem):
    wid = lax.axis_index("s") * NC + lax.axis_index("c")
    base = wid * BPW
    lanes = lax.broadcasted_iota(jnp.int32, (H,), 0)
    last = lanes == (H - 1)

    def chunk_body(c, _):
        cbase = base + c * C
        pltpu.sync_copy(uq_hbm.at[pl.ds(cbase, C)], uq)
        pltpu.sync_copy(uo_hbm.at[pl.ds(cbase, C)], uo)
        pltpu.sync_copy(iq_hbm.at[pl.ds(cbase, C)], iq)
        pltpu.sync_copy(io_hbm.at[pl.ds(cbase, C)], io)
        pltpu.sync_copy(nq_hbm.at[pl.ds(cbase * NEG, C * NEG)], nq)
        pltpu.sync_copy(no_hbm.at[pl.ds(cbase * NEG, C * NEG)],
                        no.at[pl.ds(0, C * NEG)])
        copies = [
            pltpu.async_copy(u_pack.at[uq], u_rows, sem),
            pltpu.async_copy(i_pack.at[iq], it_rows, sem),
        ]
        for k in range(NEG * C // 128):
            copies.append(pltpu.async_copy(
                i_pack.at[nq.at[pl.ds(k * 128, 128)]],
                ineg_rows.at[pl.ds(k * 128, 128)], sem))
        for cp in copies:
            cp.wait()

        def grp_body(g, _):
            uos = uo[pl.ds(g * H, H)]
            ios = io[pl.ds(g * H, H)]
            for e16 in range(H):
                e = g * H + e16
                uoe = uos[e16]
                ioe = ios[e16]
                erow = jnp.full((H,), e, jnp.int32)
                u0 = plsc.load_gather(u_rows, [erow, uoe + lanes])
                u1 = plsc.load_gather(u_rows, [erow, uoe + H + lanes])
                i0 = plsc.load_gather(it_rows, [erow, ioe + lanes])
                i1 = plsc.load_gather(it_rows, [erow, ioe + H + lanes])
                ps = plsc.cumsum(u0 * i0 + u1 * i1)
                plsc.store_scatter(
                    pos_buf, [jnp.full((H,), c * C + e, jnp.int32)], ps,
                    mask=last)
                nos = no[pl.ds(e * NEG, NEG + NEG)]
                for j in range(NEG):
                    r = e * NEG + j
                    rrow = jnp.full((H,), r, jnp.int32)
                    noj = nos[j]
                    n0 = plsc.load_gather(ineg_rows, [rrow, noj + lanes])
                    n1 = plsc.load_gather(ineg_rows, [rrow, noj + H + lanes])
                    ns = plsc.cumsum(u0 * n0 + u1 * n1)
                    plsc.store_scatter(
                        neg_buf,
                        [jnp.full((H,), (c * C + e) * NEG + j, jnp.int32)],
                        ns, mask=last)
            return 0

        lax.fori_loop(0, C // H, grp_body, 0)
        return 0

    lax.fori_loop(0, NCHUNK, chunk_body, 0)
    pltpu.sync_copy(pos_buf, pos_hbm.at[pl.ds(base, BPW)])
    pltpu.sync_copy(neg_buf, neg_hbm.at[pl.ds(base * NEG, BPW * NEG)])


_mf = pl.kernel(
    _mf_body,
    out_type=(
        jax.ShapeDtypeStruct((B,), jnp.float32),
        jax.ShapeDtypeStruct((B * NEG,), jnp.float32),
    ),
    mesh=plsc.VectorSubcoreMesh(
        core_axis_name="c", subcore_axis_name="s",
        num_cores=NC, num_subcores=NS),
    scratch_types=[
        pltpu.VMEM((C,), jnp.int32),
        pltpu.VMEM((C,), jnp.int32),
        pltpu.VMEM((C,), jnp.int32),
        pltpu.VMEM((C,), jnp.int32),
        pltpu.VMEM((C * NEG,), jnp.int32),
        pltpu.VMEM((C * NEG + H,), jnp.int32),
        pltpu.VMEM((C, 128), jnp.float32),
        pltpu.VMEM((C, 128), jnp.float32),
        pltpu.VMEM((C * NEG, 128), jnp.float32),
        pltpu.VMEM((BPW,), jnp.float32),
        pltpu.VMEM((BPW * NEG,), jnp.float32),
        pltpu.SemaphoreType.DMA,
    ],
    compiler_params=pltpu.CompilerParams(needs_layout_passes=False),
)


def kernel(user_embeds, item_embeds, users, items, items_neg):
    users = users.astype(jnp.int32)
    items = items.astype(jnp.int32)
    neg_flat = items_neg.astype(jnp.int32).reshape(B * NEG)
    uq = users >> 2
    uo = (users & 3) * 32
    iq = items >> 2
    io = (items & 3) * 32
    nq = neg_flat >> 2
    no = (neg_flat & 3) * 32
    u_tail = lax.slice(user_embeds, (NALIGNED, 0), (N, EMBED)).reshape(16, 128)
    i_tail = lax.slice(item_embeds, (NALIGNED, 0), (N, EMBED)).reshape(16, 128)
    u_pack, i_pack = _relayout(user_embeds.T, item_embeds.T, u_tail, i_tail)
    pos, neg = _mf(u_pack, i_pack, uq, uo, iq, io, nq, no)
    return pos, neg.reshape(B, NEG)


# speedup vs baseline: 2.0074x; 2.0074x over previous
import functools

import jax
import jax.numpy as jnp
from jax import lax
from jax.experimental import pallas as pl
from jax.experimental.pallas import tpu as pltpu
from jax.experimental.pallas import tpu_sc as plsc

B = 16384
EMBED = 32
NEG = 8
NC = 2
NS = 16
NW = NC * NS
BPW = B // NW
C = 128
NCHUNK = BPW // C
H = EMBED // 2


def _mf_body(user_hbm, item_hbm, users_hbm, items_hbm, negidx_hbm,
             pos_hbm, neg_hbm,
             uidx, iidx, nidx, u_rows, it_rows, ineg_rows,
             pos_buf, neg_buf, sem):
    wid = lax.axis_index("s") * NC + lax.axis_index("c")
    base = wid * BPW
    lane = lax.broadcasted_iota(jnp.int32, (H,), 0)
    last = lane == (H - 1)

    for c in range(NCHUNK):
        cbase = base + c * C
        pltpu.sync_copy(users_hbm.at[pl.ds(cbase, C)], uidx)
        pltpu.sync_copy(items_hbm.at[pl.ds(cbase, C)], iidx)
        pltpu.sync_copy(negidx_hbm.at[pl.ds(cbase * NEG, C * NEG)], nidx)
        copies = [
            pltpu.async_copy(user_hbm.at[uidx], u_rows, sem),
            pltpu.async_copy(item_hbm.at[iidx], it_rows, sem),
        ]
        for k in range(NEG):
            copies.append(pltpu.async_copy(
                item_hbm.at[nidx.at[pl.ds(k * C, C)]],
                ineg_rows.at[pl.ds(k * C, C)], sem))
        for cp in copies:
            cp.wait()

        def elem_body(e, _):
            u0 = u_rows[e, pl.ds(0, H)]
            u1 = u_rows[e, pl.ds(H, H)]
            i0 = it_rows[e, pl.ds(0, H)]
            i1 = it_rows[e, pl.ds(H, H)]
            ps = plsc.cumsum(u0 * i0 + u1 * i1)
            plsc.store_scatter(
                pos_buf, [jnp.full((H,), c * C + e, jnp.int32)], ps,
                mask=last)

            for j in range(NEG):
                r = e * NEG + j
                n0 = ineg_rows[r, pl.ds(0, H)]
                n1 = ineg_rows[r, pl.ds(H, H)]
                ns = plsc.cumsum(u0 * n0 + u1 * n1)
                plsc.store_scatter(
                    neg_buf,
                    [jnp.full((H,), (c * C + e) * NEG + j, jnp.int32)], ns,
                    mask=last)
            return 0

        lax.fori_loop(0, C, elem_body, 0, unroll=2)

    pltpu.sync_copy(pos_buf, pos_hbm.at[pl.ds(base, BPW)])
    pltpu.sync_copy(neg_buf, neg_hbm.at[pl.ds(base * NEG, BPW * NEG)])


_mf = pl.kernel(
    _mf_body,
    out_type=(
        jax.ShapeDtypeStruct((B,), jnp.float32),
        jax.ShapeDtypeStruct((B * NEG,), jnp.float32),
    ),
    mesh=plsc.VectorSubcoreMesh(
        core_axis_name="c", subcore_axis_name="s",
        num_cores=NC, num_subcores=NS),
    scratch_types=[
        pltpu.VMEM((C,), jnp.int32),
        pltpu.VMEM((C,), jnp.int32),
        pltpu.VMEM((C * NEG,), jnp.int32),
        pltpu.VMEM((C, EMBED), jnp.float32),
        pltpu.VMEM((C, EMBED), jnp.float32),
        pltpu.VMEM((C * NEG, EMBED), jnp.float32),
        pltpu.VMEM((BPW,), jnp.float32),
        pltpu.VMEM((BPW * NEG,), jnp.float32),
        pltpu.SemaphoreType.DMA,
    ],
    compiler_params=pltpu.CompilerParams(
        needs_layout_passes=False, use_tc_tiling_on_sc=False),
)


def kernel(user_embeds, item_embeds, users, items, items_neg):
    users = users.astype(jnp.int32)
    items = items.astype(jnp.int32)
    neg_flat = items_neg.astype(jnp.int32).reshape(B * NEG)
    pos, neg = _mf(user_embeds, item_embeds, users, items, neg_flat)
    return pos, neg.reshape(B, NEG)
